# baseline (speedup 1.0000x reference)
import jax
import jax.numpy as jnp
from jax.experimental import pallas as pl
from jax.experimental.pallas import tpu as pltpu


def _add_block(x_ref, p_ref, o_ref):
    o_ref[...] = x_ref[...] + p_ref[...]


def kernel(inputs, pos_table):
    B, L, D = inputs.shape
    BL = 1024

    return pl.pallas_call(
        _add_block,
        grid=(L // BL, B),
        in_specs=[
            pl.BlockSpec((1, BL, D), lambda l, b: (b, l, 0)),
            pl.BlockSpec((BL, D), lambda l, b: (l, 0)),
        ],
        out_specs=pl.BlockSpec((1, BL, D), lambda l, b: (b, l, 0)),
        out_shape=jax.ShapeDtypeStruct((B, L, D), inputs.dtype),
        compiler_params=pltpu.CompilerParams(
            dimension_semantics=("parallel", "parallel"),
        ),
    )(inputs, pos_table)
